# Initial kernel scaffold; baseline (speedup 1.0000x reference)
import jax
import jax.numpy as jnp
from jax.experimental import pallas as pl


def kernel(codes, tables):
    raise NotImplementedError("write your pallas kernel here")


# speedup vs baseline: 2.1460x; 2.1460x over previous
import jax
import jax.numpy as jnp
from jax import lax
from jax.experimental import pallas as pl
from jax.experimental.pallas import tpu as pltpu
from jax.experimental.pallas import tpu_sc as plsc

_NC = 2
_NS = 16
_NW = _NC * _NS


def _sc_body(codes_hbm, tables_hbm, out_hbm, idx_v, rows_v, out_v, sem):
    B = out_hbm.shape[0]
    N = tables_hbm.shape[0]
    C = idx_v.shape[1]
    D = tables_hbm.shape[2]
    n_per_w = B // _NW
    wid = lax.axis_index("s") * _NC + lax.axis_index("c")
    b0 = wid * n_per_w

    def batch_body(bi, carry):
        b = b0 + bi
        pltpu.sync_copy(codes_hbm.at[b], idx_v)
        for h in range(2):
            cps = [
                pltpu.async_copy(
                    tables_hbm.at[j].at[idx_v.at[2 * j + h]], rows_v.at[j], sem
                )
                for j in range(N)
            ]
            for cp in cps:
                cp.wait()

            def red(c, _):
                for d in range(D // 16):
                    acc = rows_v[0, c, pl.ds(d * 16, 16)]
                    for j in range(1, N):
                        acc = acc + rows_v[j, c, pl.ds(d * 16, 16)]
                    out_v[h * C + c, pl.ds(d * 16, 16)] = acc
                return 0

            lax.fori_loop(0, C, red, 0)
        pltpu.sync_copy(out_v, out_hbm.at[b])
        return carry

    lax.fori_loop(0, n_per_w, batch_body, 0)


def kernel(codes, tables):
    B, N, L = codes.shape
    D = tables.shape[-1]
    C = L // 2
    codes3 = codes.reshape(B, 2 * N, C)

    k = pl.kernel(
        _sc_body,
        out_type=jax.ShapeDtypeStruct((B, L, D), tables.dtype),
        mesh=plsc.VectorSubcoreMesh(core_axis_name="c", subcore_axis_name="s"),
        compiler_params=pltpu.CompilerParams(use_tc_tiling_on_sc=False),
        scratch_types=[
            pltpu.VMEM((2 * N, C), jnp.int32),
            pltpu.VMEM((N, C, D), jnp.float32),
            pltpu.VMEM((L, D), jnp.float32),
            pltpu.SemaphoreType.DMA,
        ],
    )
    return k(codes3, tables)


# speedup vs baseline: 2.2599x; 1.0531x over previous
import jax
import jax.numpy as jnp
from jax import lax
from jax.experimental import pallas as pl
from jax.experimental.pallas import tpu as pltpu
from jax.experimental.pallas import tpu_sc as plsc

_NC = 2
_NS = 16
_NW = _NC * _NS


def _sc_body(codes_hbm, tables_hbm, out_hbm, idx_v, rows_v, out_v,
             isem, gsem0, gsem1):
    B = out_hbm.shape[0]
    N = tables_hbm.shape[0]
    C = idx_v.shape[2]
    D = tables_hbm.shape[2]
    n_per_w = B // _NW
    wid = lax.axis_index("s") * _NC + lax.axis_index("c")
    b0 = wid * n_per_w
    gsems = (gsem0, gsem1)

    def fire_gathers(slot, h, buf):
        for j in range(N):
            pltpu.async_copy(
                tables_hbm.at[j].at[idx_v.at[slot, 2 * j + h]],
                rows_v.at[buf, j], gsems[buf])

    def drain_gathers(buf):
        pltpu.make_async_copy(
            tables_hbm.at[pl.ds(0, N), pl.ds(0, C)], rows_v.at[buf],
            gsems[buf]).wait()

    def fire_idx(bi, slot):
        pltpu.async_copy(codes_hbm.at[b0 + bi], idx_v.at[slot], isem)

    def wait_idx(slot):
        pltpu.make_async_copy(codes_hbm.at[0], idx_v.at[slot], isem).wait()

    def reduce_half(buf, h):
        def red(c, _):
            for d in range(D // 16):
                acc = rows_v[buf, 0, c, pl.ds(d * 16, 16)]
                for j in range(1, N):
                    acc = acc + rows_v[buf, j, c, pl.ds(d * 16, 16)]
                out_v[h * C + c, pl.ds(d * 16, 16)] = acc
            return 0
        lax.fori_loop(0, C, red, 0, unroll=2)

    pltpu.sync_copy(codes_hbm.at[b0], idx_v.at[0])
    fire_gathers(0, 0, 0)
    fire_idx(1, 1)
    fire_idx(2, 2)

    def batch_body(bi, carry):
        b = b0 + bi
        slot = lax.rem(bi, 4)
        drain_gathers(0)
        fire_gathers(slot, 1, 1)
        reduce_half(0, 0)
        drain_gathers(1)

        @pl.when(bi < n_per_w - 1)
        def _():
            wait_idx(lax.rem(bi + 1, 4))
            fire_gathers(lax.rem(bi + 1, 4), 0, 0)

            @pl.when(bi + 3 < n_per_w)
            def _():
                fire_idx(bi + 3, lax.rem(bi + 3, 4))

        reduce_half(1, 1)
        pltpu.sync_copy(out_v, out_hbm.at[b])
        return carry

    lax.fori_loop(0, n_per_w, batch_body, 0)


def kernel(codes, tables):
    B, N, L = codes.shape
    D = tables.shape[-1]
    C = L // 2
    codes3 = codes.reshape(B, 2 * N, C)

    k = pl.kernel(
        _sc_body,
        out_type=jax.ShapeDtypeStruct((B, L, D), tables.dtype),
        mesh=plsc.VectorSubcoreMesh(core_axis_name="c", subcore_axis_name="s"),
        compiler_params=pltpu.CompilerParams(use_tc_tiling_on_sc=False),
        scratch_types=[
            pltpu.VMEM((4, 2 * N, C), jnp.int32),
            pltpu.VMEM((2, N, C, D), jnp.float32),
            pltpu.VMEM((L, D), jnp.float32),
            pltpu.SemaphoreType.DMA,
            pltpu.SemaphoreType.DMA,
            pltpu.SemaphoreType.DMA,
        ],
    )
    return k(codes3, tables)
